# baseline (speedup 1.0000x reference)
import jax
import jax.numpy as jnp
from jax import lax
from jax.experimental import pallas as pl

N_OBJ = 128
T = 100
D = 64
ROWS = 128


NB = 64


def _concat_body(ids_ref, s_ref, tab_ref, o_ref):
    ids = ids_ref[...]
    cols = lax.broadcasted_iota(jnp.int32, (NB, N_OBJ), 1)
    onehot = (ids == cols).astype(jnp.float32)
    emb = jnp.dot(onehot, tab_ref[...],
                  preferred_element_type=jnp.float32)
    embb = jnp.broadcast_to(emb[None, :, None, :], (1, NB, T, D))
    o_ref[...] = jnp.concatenate([s_ref[...], embb], axis=-1)


def kernel(states, table, obj_ids):
    B, N, t, d = states.shape
    ids2d = obj_ids.reshape(N, 1)
    return pl.pallas_call(
        _concat_body,
        grid=(B, N // NB),
        in_specs=[
            pl.BlockSpec((NB, 1), lambda g, h: (h, 0)),
            pl.BlockSpec((1, NB, t, d), lambda g, h: (g, h, 0, 0)),
            pl.BlockSpec((N_OBJ, d), lambda g, h: (0, 0)),
        ],
        out_specs=pl.BlockSpec((1, NB, t, 2 * d), lambda g, h: (g, h, 0, 0)),
        out_shape=jax.ShapeDtypeStruct((B, N, t, 2 * d), jnp.float32),
    )(ids2d, states, table)


# speedup vs baseline: 2.5311x; 2.5311x over previous
import jax
import jax.numpy as jnp
from jax import lax
from jax.experimental import pallas as pl

N_OBJ = 128
D = 64
TB = 25


def _concat_body(ids_ref, s_ref, tab_ref, o_ref):
    ids = ids_ref[...]
    cols = lax.broadcasted_iota(jnp.int32, (N_OBJ, N_OBJ), 1)
    onehot = (ids == cols).astype(jnp.float32)
    emb = jnp.dot(onehot, tab_ref[...],
                  preferred_element_type=jnp.float32)
    st = jnp.swapaxes(s_ref[0], 1, 2)
    embb = jnp.broadcast_to(emb[None], (TB, N_OBJ, D))
    o_ref[0] = jnp.concatenate([st, embb], axis=-1)


def kernel(states, table, obj_ids):
    B, N, t, d = states.shape
    ids2d = obj_ids.reshape(N, 1)
    s_t = states.transpose(0, 2, 3, 1)
    out_t = pl.pallas_call(
        _concat_body,
        grid=(B, t // TB),
        in_specs=[
            pl.BlockSpec((N, 1), lambda g, h: (0, 0)),
            pl.BlockSpec((1, TB, d, N), lambda g, h: (g, h, 0, 0)),
            pl.BlockSpec((N, d), lambda g, h: (0, 0)),
        ],
        out_specs=pl.BlockSpec((1, TB, N, 2 * d), lambda g, h: (g, h, 0, 0)),
        out_shape=jax.ShapeDtypeStruct((B, t, N, 2 * d), jnp.float32),
    )(ids2d, s_t, table)
    return out_t.transpose(0, 2, 1, 3)


# speedup vs baseline: 3.1886x; 1.2598x over previous
import jax
import jax.numpy as jnp
from jax import lax
from jax.experimental import pallas as pl

N_OBJ = 128
D = 64
TB = 50


def _concat_body(ids_ref, s_ref, tab_ref, o_ref):
    ids = ids_ref[...]
    cols = lax.broadcasted_iota(jnp.int32, (N_OBJ, N_OBJ), 1)
    onehot = (ids == cols).astype(jnp.float32)
    emb = jnp.dot(onehot, tab_ref[...],
                  preferred_element_type=jnp.float32)
    st = jnp.swapaxes(s_ref[0], 1, 2)
    embb = jnp.broadcast_to(emb[None], (TB, N_OBJ, D))
    o_ref[0] = jnp.concatenate([st, embb], axis=-1)


def kernel(states, table, obj_ids):
    B, N, t, d = states.shape
    ids2d = obj_ids.reshape(N, 1)
    s_t = states.transpose(0, 2, 3, 1)
    out_t = pl.pallas_call(
        _concat_body,
        grid=(B, t // TB),
        in_specs=[
            pl.BlockSpec((N, 1), lambda g, h: (0, 0)),
            pl.BlockSpec((1, TB, d, N), lambda g, h: (g, h, 0, 0)),
            pl.BlockSpec((N, d), lambda g, h: (0, 0)),
        ],
        out_specs=pl.BlockSpec((1, TB, N, 2 * d), lambda g, h: (g, h, 0, 0)),
        out_shape=jax.ShapeDtypeStruct((B, t, N, 2 * d), jnp.float32),
    )(ids2d, s_t, table)
    return out_t.transpose(0, 2, 1, 3)


# speedup vs baseline: 3.7079x; 1.1629x over previous
import jax
import jax.numpy as jnp
from jax import lax
from jax.experimental import pallas as pl

N_OBJ = 128
D = 64
TB = 100


def _concat_body(ids_ref, s_ref, tab_ref, o_ref):
    ids = ids_ref[...]
    cols = lax.broadcasted_iota(jnp.int32, (N_OBJ, N_OBJ), 1)
    onehot = (ids == cols).astype(jnp.float32)
    emb = jnp.dot(onehot, tab_ref[...],
                  preferred_element_type=jnp.float32)
    st = jnp.swapaxes(s_ref[0], 1, 2)
    embb = jnp.broadcast_to(emb[None], (TB, N_OBJ, D))
    o_ref[0] = jnp.concatenate([st, embb], axis=-1)


def kernel(states, table, obj_ids):
    B, N, t, d = states.shape
    ids2d = obj_ids.reshape(N, 1)
    s_t = states.transpose(0, 2, 3, 1)
    out_t = pl.pallas_call(
        _concat_body,
        grid=(B, t // TB),
        in_specs=[
            pl.BlockSpec((N, 1), lambda g, h: (0, 0)),
            pl.BlockSpec((1, TB, d, N), lambda g, h: (g, h, 0, 0)),
            pl.BlockSpec((N, d), lambda g, h: (0, 0)),
        ],
        out_specs=pl.BlockSpec((1, TB, N, 2 * d), lambda g, h: (g, h, 0, 0)),
        out_shape=jax.ShapeDtypeStruct((B, t, N, 2 * d), jnp.float32),
    )(ids2d, s_t, table)
    return out_t.transpose(0, 2, 1, 3)


# speedup vs baseline: 3.9678x; 1.0701x over previous
import jax
import jax.numpy as jnp
from jax import lax
from jax.experimental import pallas as pl

N_OBJ = 128
D = 64
TB = 100
BB = 2


def _concat_body(ids_ref, s_ref, tab_ref, o_ref):
    ids = ids_ref[...]
    cols = lax.broadcasted_iota(jnp.int32, (N_OBJ, N_OBJ), 1)
    onehot = (ids == cols).astype(jnp.float32)
    emb = jnp.dot(onehot, tab_ref[...],
                  preferred_element_type=jnp.float32)
    st = jnp.swapaxes(s_ref[...], 2, 3)
    embb = jnp.broadcast_to(emb[None, None], (BB, TB, N_OBJ, D))
    o_ref[...] = jnp.concatenate([st, embb], axis=-1)


def kernel(states, table, obj_ids):
    B, N, t, d = states.shape
    ids2d = obj_ids.reshape(N, 1)
    s_t = states.transpose(0, 2, 3, 1)
    out_t = pl.pallas_call(
        _concat_body,
        grid=(B // BB, t // TB),
        in_specs=[
            pl.BlockSpec((N, 1), lambda g, h: (0, 0)),
            pl.BlockSpec((BB, TB, d, N), lambda g, h: (g, h, 0, 0)),
            pl.BlockSpec((N, d), lambda g, h: (0, 0)),
        ],
        out_specs=pl.BlockSpec((BB, TB, N, 2 * d), lambda g, h: (g, h, 0, 0)),
        out_shape=jax.ShapeDtypeStruct((B, t, N, 2 * d), jnp.float32),
    )(ids2d, s_t, table)
    return out_t.transpose(0, 2, 1, 3)


# speedup vs baseline: 3.9960x; 1.0071x over previous
import jax
import jax.numpy as jnp
from jax import lax
from jax.experimental import pallas as pl

N_OBJ = 128
D = 64
TB = 100
BB = 2


def _concat_body(ids_ref, s_ref, tab_ref, o_ref):
    ids = ids_ref[...]
    cols = lax.broadcasted_iota(jnp.int32, (N_OBJ, N_OBJ), 1)
    onehot = (ids == cols).astype(jnp.float32)
    emb = jnp.dot(onehot, tab_ref[...],
                  preferred_element_type=jnp.float32)
    st = jnp.swapaxes(s_ref[...], 2, 3)
    embb = jnp.broadcast_to(emb[None, None], (BB, TB, N_OBJ, D))
    o_ref[:, :, :, 0:D] = st
    o_ref[:, :, :, D:2 * D] = embb


def kernel(states, table, obj_ids):
    B, N, t, d = states.shape
    ids2d = obj_ids.reshape(N, 1)
    s_t = states.transpose(0, 2, 3, 1)
    out_t = pl.pallas_call(
        _concat_body,
        grid=(B // BB, t // TB),
        in_specs=[
            pl.BlockSpec((N, 1), lambda g, h: (0, 0)),
            pl.BlockSpec((BB, TB, d, N), lambda g, h: (g, h, 0, 0)),
            pl.BlockSpec((N, d), lambda g, h: (0, 0)),
        ],
        out_specs=pl.BlockSpec((BB, TB, N, 2 * d), lambda g, h: (g, h, 0, 0)),
        out_shape=jax.ShapeDtypeStruct((B, t, N, 2 * d), jnp.float32),
    )(ids2d, s_t, table)
    return out_t.transpose(0, 2, 1, 3)
